# speedup vs baseline: 2.2329x; 2.2329x over previous
import functools

import jax
import jax.numpy as jnp
from jax.experimental import pallas as pl
from jax.experimental.pallas import tpu as pltpu

_TAU0 = 0.5
_K = 16


def _body(logits_ref, uniform_ref, out_ref):
    l = logits_ref[...]
    m = jnp.max(l, axis=-1, keepdims=True)
    e = jnp.exp(2.0 * (l - m))
    u = uniform_ref[...]
    v = -jnp.log(jnp.maximum(u, 1e-10))
    n = e[:, None, :] / (v * v)
    s = jnp.sum(n, axis=-1, keepdims=True)
    out_ref[...] = jnp.max(n * (1.0 / s), axis=1)


@jax.jit
def kernel(logits, uniform):
    B, D = logits.shape
    K = uniform.shape[1]
    bb = 8
    grid = (B // bb,)
    return pl.pallas_call(
        _body,
        grid=grid,
        in_specs=[
            pl.BlockSpec((bb, D), lambda i: (i, 0)),
            pl.BlockSpec((bb, K, D), lambda i: (i, 0, 0)),
        ],
        out_specs=pl.BlockSpec((bb, D), lambda i: (i, 0)),
        out_shape=jax.ShapeDtypeStruct((B, D), logits.dtype),
        compiler_params=pltpu.CompilerParams(
            dimension_semantics=("arbitrary",),
        ),
    )(logits, uniform)
